# baseline (speedup 1.0000x reference)
import functools

import jax
import jax.numpy as jnp
from jax import lax
from jax.experimental import pallas as pl
from jax.experimental.pallas import tpu as pltpu
from jax.experimental.pallas import tpu_sc as plsc

_S, _K, _A, _T, _C = 32, 64, 64, 80, 3
_KP = 6
_NMS_THRESH = (2.5, 1.0, 2.0)
_ROW = _T * _C
_KEEP = 16 * _C
_B = _S * _A * _KP
_NW = 32
_R = _S * _K * _A
_NSEG = _R * _ROW // 128
_RW = _R // _NW
_ECH = _RW // 128
_GCH = 12
_GSEG = 96
_GROWS = _GSEG // 3
_RING = 4

_SC_PARAMS = pltpu.CompilerParams(
    use_tc_tiling_on_sc=True, needs_layout_passes=False)


def _sc_extract_body(table_hbm, out_hbm, idx_v, ring_v, xy_v, *sems):
    wid = lax.axis_index("s") * 2 + lax.axis_index("c")
    lane = lax.iota(jnp.int32, 16)
    base = wid * _RW
    for ch in range(_ECH):
        for u in range(8):
            r = base + ch * 128 + u * 16 + lane
            idx_v[ch, pl.ds(u * 16, 16)] = (r * 240 + 237) >> 7
    copies = {}
    for ch in range(_RING):
        copies[ch] = pltpu.async_copy(
            table_hbm.at[idx_v.at[ch]], ring_v.at[ch], sems[ch])
    patx = (((lane & 7) * 240 + 237) & 127)
    for ch in range(_ECH):
        copies.pop(ch).wait()
        b = ch % _RING
        bf = jnp.full((16,), b, jnp.int32)
        for u in range(8):
            rowv = u * 16 + lane
            xy_v[0, ch, pl.ds(u * 16, 16)] = plsc.load_gather(
                ring_v, [bf, rowv, patx])
            xy_v[1, ch, pl.ds(u * 16, 16)] = plsc.load_gather(
                ring_v, [bf, rowv, patx + 1])
        nxt = ch + _RING
        if nxt < _ECH:
            copies[nxt] = pltpu.async_copy(
                table_hbm.at[idx_v.at[nxt]], ring_v.at[b], sems[b])
    pltpu.sync_copy(xy_v, out_hbm.at[wid])


def _nms_body(sc_ref, xs_ref, ys_ref, agt_ref, seg_ref, sout_ref):
    s = pl.program_id(0)
    sc_raw = sc_ref[0]
    xs = xs_ref[0]
    ys = ys_ref[0]
    agt = agt_ref[0]
    thresh = (_NMS_THRESH[0] * agt[0:1, :]
              + _NMS_THRESH[1] * agt[1:2, :]
              + _NMS_THRESH[2] * agt[2:3, :])

    m = jnp.max(sc_raw, axis=0, keepdims=True)
    e = jnp.exp(sc_raw - m)
    p = e / jnp.sum(e, axis=0, keepdims=True)

    kiota = lax.broadcasted_iota(jnp.int32, (_K, _A), 0)
    aiota = lax.broadcasted_iota(jnp.int32, (1, _A), 1)

    scn = p
    psel = []
    for j in range(_KP):
        mx = jnp.max(scn, axis=0, keepdims=True)
        idx = jnp.min(jnp.where(scn == mx, kiota, _K), axis=0, keepdims=True)
        oh = kiota == idx
        xsel = jnp.sum(jnp.where(oh, xs, 0.0), axis=0, keepdims=True)
        ysel = jnp.sum(jnp.where(oh, ys, 0.0), axis=0, keepdims=True)
        psel.append(jnp.sum(jnp.where(oh, p, 0.0), axis=0, keepdims=True))
        dx = xs - xsel
        dy = ys - ysel
        drow = jnp.sqrt(dx * dx + dy * dy)
        within = drow < thresh
        scn = scn * jnp.where(within, 0.01, 1.0)
        scn = jnp.where(oh, -1.0, scn)
        r = s * (_K * _A) + idx * _A + aiota
        seg0 = lax.shift_right_logical(r * 15, 3)
        for v in range(3):
            seg_ref[0, 3 * j + v:3 * j + v + 1, :] = jnp.minimum(
                seg0 + v, _NSEG - 1)

    pm = psel[0]
    for j in range(1, _KP):
        pm = jnp.maximum(pm, psel[j])
    r2 = [(pj / pm) * (pj / pm) for pj in psel]
    tot = r2[0]
    for j in range(1, _KP):
        tot = tot + r2[j]
    for j in range(_KP):
        sout_ref[0, j:j + 1, :] = r2[j] / tot


_nms_call = pl.pallas_call(
    _nms_body,
    grid=(_S,),
    in_specs=[
        pl.BlockSpec((1, _K, _A), lambda s: (s, 0, 0)),
        pl.BlockSpec((1, _K, _A), lambda s: (s, 0, 0)),
        pl.BlockSpec((1, _K, _A), lambda s: (s, 0, 0)),
        pl.BlockSpec((1, _C, _A), lambda s: (s, 0, 0)),
    ],
    out_specs=[
        pl.BlockSpec((1, 3 * _KP, _A), lambda s: (s, 0, 0)),
        pl.BlockSpec((1, _KP, _A), lambda s: (s, 0, 0)),
    ],
    out_shape=[
        jax.ShapeDtypeStruct((_S, 3 * _KP, _A), jnp.int32),
        jax.ShapeDtypeStruct((_S, _KP, _A), jnp.float32),
    ],
)


def _sc_gather_body(table_hbm, seg_hbm, out_hbm, idx_v, ring_v, out_v, *sems):
    wid = lax.axis_index("s") * 2 + lax.axis_index("c")
    lane = lax.iota(jnp.int32, 16)
    pltpu.sync_copy(seg_hbm.at[wid], idx_v)
    copies = {}
    for ch in range(_RING):
        copies[ch] = pltpu.async_copy(
            table_hbm.at[idx_v.at[ch]], ring_v.at[ch], sems[ch])

    for ch in range(_GCH):
        copies.pop(ch).wait()
        b = ch % _RING
        bf = jnp.full((16,), b, jnp.int32)
        for g in range(_GROWS // 16):
            il = g * 16 + lane
            pos = wid * (_B // _NW) + ch * _GROWS + il
            a = (pos // _KP) % _A
            off = ((0 - a) & 7) * 16
            rowbase = il * (3 * 128) + off
            outbase = (ch * _GROWS + il) * _KEEP

            def tbody(t5, carry, rowbase=rowbase, outbase=outbase, bf=bf):
                src0 = rowbase + (12 + 15 * t5)
                dst0 = outbase + 3 * t5
                for c3 in range(_C):
                    sp = src0 + c3
                    dp = dst0 + c3
                    gval = plsc.load_gather(
                        ring_v,
                        [bf, lax.shift_right_logical(sp, 7), sp & 127])
                    plsc.store_scatter(
                        out_v,
                        [lax.shift_right_logical(dp, 7), dp & 127], gval)
                return carry

            lax.fori_loop(0, 16, tbody, 0)
        nxt = ch + _RING
        if nxt < _GCH:
            copies[nxt] = pltpu.async_copy(
                table_hbm.at[idx_v.at[nxt]], ring_v.at[b], sems[b])

    pltpu.sync_copy(out_v, out_hbm.at[wid])


@functools.cache
def _sc_calls():
    mesh = plsc.VectorSubcoreMesh(core_axis_name="c", subcore_axis_name="s")
    extract = functools.partial(
        pl.kernel,
        mesh=mesh,
        out_type=jax.ShapeDtypeStruct((_NW, 2, _ECH, 128), jnp.float32),
        compiler_params=_SC_PARAMS,
        scratch_types=[
            pltpu.VMEM((_ECH, 128), jnp.int32),
            pltpu.VMEM((_RING, 128, 128), jnp.float32),
            pltpu.VMEM((2, _ECH, 128), jnp.float32),
        ] + [pltpu.SemaphoreType.DMA] * _RING,
    )(_sc_extract_body)
    gather = functools.partial(
        pl.kernel,
        mesh=mesh,
        out_type=jax.ShapeDtypeStruct((_NW, _B // _NW * _KEEP // 128, 128),
                                      jnp.float32),
        compiler_params=_SC_PARAMS,
        scratch_types=[
            pltpu.VMEM((_GCH, _GSEG), jnp.int32),
            pltpu.VMEM((_RING, _GSEG, 128), jnp.float32),
            pltpu.VMEM((_B // _NW * _KEEP // 128, 128), jnp.float32),
        ] + [pltpu.SemaphoreType.DMA] * _RING,
    )(_sc_gather_body)
    return extract, gather


def kernel(ag_type, trajs, scores):
    extract, gather = _sc_calls()
    table = trajs.reshape(_NSEG, 128)
    xy = extract(table)
    xs = xy[:, 0].reshape(_S, _K, _A)
    ys = xy[:, 1].reshape(_S, _K, _A)
    agt = jnp.swapaxes(ag_type, 1, 2)
    segs, sout = _nms_call(scores, xs, ys, agt)
    scores_k = jnp.swapaxes(sout, 1, 2)
    seg_idx = jnp.transpose(segs, (0, 2, 1)).reshape(_NW, _GCH, _GSEG)
    rows = gather(table, seg_idx)
    trajs_out = rows.reshape(_S, _A, _KP, 16, _C)
    return trajs_out, scores_k


# speedup vs baseline: 73.2250x; 73.2250x over previous
import jax
import jax.numpy as jnp
from jax import lax
from jax.experimental import pallas as pl

_S, _K, _A, _T, _C = 32, 64, 64, 80, 3
_KP = 6
_NMS_THRESH = (2.5, 1.0, 2.0)
_T16 = 16


def _nms_body(sc_ref, tr_ref, agt_ref, sel_ref, sout_ref):
    sc_raw = sc_ref[0]
    agt = agt_ref[0]
    thresh = (_NMS_THRESH[0] * agt[0:1, :]
              + _NMS_THRESH[1] * agt[1:2, :]
              + _NMS_THRESH[2] * agt[2:3, :])

    tiota = lax.broadcasted_iota(jnp.int32, (_K, _A, _T), 2)
    xs = jnp.sum(jnp.where(tiota == _T - 1, tr_ref[0, :, 0, :, :], 0.0), axis=2)
    ys = jnp.sum(jnp.where(tiota == _T - 1, tr_ref[0, :, 1, :, :], 0.0), axis=2)

    m = jnp.max(sc_raw, axis=0, keepdims=True)
    e = jnp.exp(sc_raw - m)
    p = e / jnp.sum(e, axis=0, keepdims=True)

    kiota = lax.broadcasted_iota(jnp.int32, (_K, _A), 0)

    scn = p
    idxs = []
    psel = []
    for j in range(_KP):
        mx = jnp.max(scn, axis=0, keepdims=True)
        idx = jnp.min(jnp.where(scn == mx, kiota, _K), axis=0, keepdims=True)
        idxs.append(idx)
        oh = kiota == idx
        xsel = jnp.sum(jnp.where(oh, xs, 0.0), axis=0, keepdims=True)
        ysel = jnp.sum(jnp.where(oh, ys, 0.0), axis=0, keepdims=True)
        psel.append(jnp.sum(jnp.where(oh, p, 0.0), axis=0, keepdims=True))
        dx = xs - xsel
        dy = ys - ysel
        drow = jnp.sqrt(dx * dx + dy * dy)
        within = drow < thresh
        scn = scn * jnp.where(within, 0.01, 1.0)
        scn = jnp.where(oh, -1.0, scn)

    onesT = jnp.full((1, _T), 1.0, jnp.float32)
    kiota3 = lax.broadcasted_iota(jnp.int32, (_K, _A, _T), 0)
    tsel = (lax.broadcasted_iota(jnp.int32, (_T, _T16), 0)
            == 4 + 5 * lax.broadcasted_iota(jnp.int32, (_T, _T16), 1))
    tself = tsel.astype(jnp.float32)
    for j in range(_KP):
        kvmat = jax.lax.dot_general(
            idxs[j].astype(jnp.float32), onesT, (((0,), (0,)), ((), ())),
            preferred_element_type=jnp.float32)
        kv3 = lax.broadcast_in_dim(kvmat.astype(jnp.int32), (_K, _A, _T), (1, 2))
        mask3 = kiota3 == kv3
        for c in range(_C):
            plane = tr_ref[0, :, c, :, :]
            selc = jnp.sum(jnp.where(mask3, plane, 0.0), axis=0)
            sel_ref[0, j, c] = jax.lax.dot_general(
                selc, tself, (((1,), (0,)), ((), ())),
                preferred_element_type=jnp.float32)

    pm = psel[0]
    for j in range(1, _KP):
        pm = jnp.maximum(pm, psel[j])
    r2 = [(pj / pm) * (pj / pm) for pj in psel]
    tot = r2[0]
    for j in range(1, _KP):
        tot = tot + r2[j]
    for j in range(_KP):
        sout_ref[0, j:j + 1, :] = r2[j] / tot


_nms_call = pl.pallas_call(
    _nms_body,
    grid=(_S,),
    in_specs=[
        pl.BlockSpec((1, _K, _A), lambda s: (s, 0, 0)),
        pl.BlockSpec((1, _K, _C, _A, _T), lambda s: (s, 0, 0, 0, 0)),
        pl.BlockSpec((1, _C, _A), lambda s: (s, 0, 0)),
    ],
    out_specs=[
        pl.BlockSpec((1, _KP, _C, _A, _T16), lambda s: (s, 0, 0, 0, 0)),
        pl.BlockSpec((1, _KP, _A), lambda s: (s, 0, 0)),
    ],
    out_shape=[
        jax.ShapeDtypeStruct((_S, _KP, _C, _A, _T16), jnp.float32),
        jax.ShapeDtypeStruct((_S, _KP, _A), jnp.float32),
    ],
)


def kernel(ag_type, trajs, scores):
    trajs_t = jnp.transpose(trajs, (0, 1, 4, 2, 3))
    agt = jnp.swapaxes(ag_type, 1, 2)
    sel, sout = _nms_call(scores, trajs_t, agt)
    scores_k = jnp.swapaxes(sout, 1, 2)
    trajs_out = jnp.transpose(sel, (0, 3, 1, 4, 2))
    return trajs_out, scores_k


# speedup vs baseline: 73.2948x; 1.0010x over previous
import jax
import jax.numpy as jnp
from jax import lax
from jax.experimental import pallas as pl

_S, _K, _A, _T, _C = 32, 64, 64, 80, 3
_KP = 6
_NMS_THRESH = (2.5, 1.0, 2.0)
_T16 = 16


def _nms_body(sc_ref, tr_ref, agt_ref, sel_ref, sout_ref):
    sc_raw = sc_ref[0]
    agt = agt_ref[0]
    thresh = (_NMS_THRESH[0] * agt[0:1, :]
              + _NMS_THRESH[1] * agt[1:2, :]
              + _NMS_THRESH[2] * agt[2:3, :])

    tiota = lax.broadcasted_iota(jnp.int32, (_K, _A, _T), 2)
    xs = jnp.sum(jnp.where(tiota == _T - 1, tr_ref[0, :, 0, :, :], 0.0), axis=2)
    ys = jnp.sum(jnp.where(tiota == _T - 1, tr_ref[0, :, 1, :, :], 0.0), axis=2)

    m = jnp.max(sc_raw, axis=0, keepdims=True)
    e = jnp.exp(sc_raw - m)
    p = e / jnp.sum(e, axis=0, keepdims=True)

    kiota = lax.broadcasted_iota(jnp.int32, (_K, _A), 0)

    scn = p
    idxs = []
    psel = []
    for j in range(_KP):
        mx = jnp.max(scn, axis=0, keepdims=True)
        idx = jnp.min(jnp.where(scn == mx, kiota, _K), axis=0, keepdims=True)
        idxs.append(idx)
        oh = kiota == idx
        xsel = jnp.sum(jnp.where(oh, xs, 0.0), axis=0, keepdims=True)
        ysel = jnp.sum(jnp.where(oh, ys, 0.0), axis=0, keepdims=True)
        psel.append(jnp.sum(jnp.where(oh, p, 0.0), axis=0, keepdims=True))
        dx = xs - xsel
        dy = ys - ysel
        drow = jnp.sqrt(dx * dx + dy * dy)
        within = drow < thresh
        scn = scn * jnp.where(within, 0.01, 1.0)
        scn = jnp.where(oh, -1.0, scn)

    onesT = jnp.full((1, _T), 1.0, jnp.float32)
    kiota3 = lax.broadcasted_iota(jnp.int32, (_K, _A, _T), 0)
    tsel = (lax.broadcasted_iota(jnp.int32, (_T, _T16), 0)
            == 4 + 5 * lax.broadcasted_iota(jnp.int32, (_T, _T16), 1))
    tself = tsel.astype(jnp.float32)
    for j in range(_KP):
        kvmat = jax.lax.dot_general(
            idxs[j].astype(jnp.float32), onesT, (((0,), (0,)), ((), ())),
            precision=jax.lax.Precision.HIGHEST,
            preferred_element_type=jnp.float32)
        kv3 = lax.broadcast_in_dim(kvmat.astype(jnp.int32), (_K, _A, _T), (1, 2))
        mask3 = kiota3 == kv3
        for c in range(_C):
            plane = tr_ref[0, :, c, :, :]
            selc = jnp.sum(jnp.where(mask3, plane, 0.0), axis=0)
            sel_ref[0, j, c] = jax.lax.dot_general(
                selc, tself, (((1,), (0,)), ((), ())),
                precision=jax.lax.Precision.HIGHEST,
                preferred_element_type=jnp.float32)

    pm = psel[0]
    for j in range(1, _KP):
        pm = jnp.maximum(pm, psel[j])
    r2 = [(pj / pm) * (pj / pm) for pj in psel]
    tot = r2[0]
    for j in range(1, _KP):
        tot = tot + r2[j]
    for j in range(_KP):
        sout_ref[0, j:j + 1, :] = r2[j] / tot


_nms_call = pl.pallas_call(
    _nms_body,
    grid=(_S,),
    in_specs=[
        pl.BlockSpec((1, _K, _A), lambda s: (s, 0, 0)),
        pl.BlockSpec((1, _K, _C, _A, _T), lambda s: (s, 0, 0, 0, 0)),
        pl.BlockSpec((1, _C, _A), lambda s: (s, 0, 0)),
    ],
    out_specs=[
        pl.BlockSpec((1, _KP, _C, _A, _T16), lambda s: (s, 0, 0, 0, 0)),
        pl.BlockSpec((1, _KP, _A), lambda s: (s, 0, 0)),
    ],
    out_shape=[
        jax.ShapeDtypeStruct((_S, _KP, _C, _A, _T16), jnp.float32),
        jax.ShapeDtypeStruct((_S, _KP, _A), jnp.float32),
    ],
)


def kernel(ag_type, trajs, scores):
    trajs_t = jnp.transpose(trajs, (0, 1, 4, 2, 3))
    agt = jnp.swapaxes(ag_type, 1, 2)
    sel, sout = _nms_call(scores, trajs_t, agt)
    scores_k = jnp.swapaxes(sout, 1, 2)
    trajs_out = jnp.transpose(sel, (0, 3, 1, 4, 2))
    return trajs_out, scores_k
